# baseline (speedup 1.0000x reference)
import functools
import math

import jax
import jax.numpy as jnp
from jax import lax
from jax.experimental import pallas as pl
from jax.experimental.pallas import tpu as pltpu
from jax.experimental.pallas import tpu_sc as plsc

_B, _M, _T, _HZ = 32, 8, 128, 4
_D, _HID, _K = 512, 1024, 8
_G = _B * _M
_R = _G * _HZ
_GPB = 64
_STEPS = _G // _GPB
_RC = 256


def _dot(a, b):
    return jax.lax.dot_general(a, b, (((1,), (0,)), ((), ())),
                               preferred_element_type=jnp.float32)


def _gelu(x):
    return 0.5 * x * (1.0 + lax.erf(x * (1.0 / math.sqrt(2.0))))


def _topk_body(sel_ref, conf_ref, agree_ref, idx_ref, gidx_ref, w_ref,
               stats_ref):
    sel = sel_ref[...]
    conf = jnp.clip(conf_ref[...], 0.0, 1.0)
    agree = jnp.clip(agree_ref[...], 0.0, 1.0)
    iota = lax.broadcasted_iota(jnp.int32, (_R, _T), 1)
    row = sel
    idxs, vals, confs, agrees = [], [], [], []
    for _ in range(_K):
        m = jnp.max(row, axis=1, keepdims=True)
        is_max = row == m
        idx_j = jnp.min(jnp.where(is_max, iota, _T), axis=1, keepdims=True)
        pick = iota == idx_j
        confs.append(jnp.sum(jnp.where(pick, conf, 0.0), axis=1,
                             keepdims=True))
        agrees.append(jnp.sum(jnp.where(pick, agree, 0.0), axis=1,
                              keepdims=True))
        row = jnp.where(pick, -jnp.inf, row)
        idxs.append(idx_j)
        vals.append(m)
    idx = jnp.concatenate(idxs, axis=1)
    val = jnp.concatenate(vals, axis=1)
    tconf = jnp.concatenate(confs, axis=1)
    tagree = jnp.concatenate(agrees, axis=1)
    w = val * jnp.clip(tconf, 0.05, 1.0) * jnp.clip(tagree, 0.05, 1.0)
    w = w / jnp.clip(jnp.sum(w, axis=1, keepdims=True), 1e-6)
    maxw = jnp.max(w, axis=1, keepdims=True)
    wc = jnp.clip(w, 1e-8)
    ent = (-jnp.sum(wc * jnp.log(wc), axis=1, keepdims=True)
           / math.log(float(max(_K, 2))))
    cm = jnp.sum(w * tconf, axis=1, keepdims=True)
    am = jnp.sum(w * tagree, axis=1, keepdims=True)
    idx_ref[...] = idx
    r_io = lax.broadcasted_iota(jnp.int32, (_R, _K), 0)
    gidx_ref[...] = idx + (r_io // _HZ) * _T
    w_ref[...] = w
    stats_ref[...] = jnp.concatenate([maxw, ent, cm, am], axis=1)


_NW = 32
_ROWS = _R * _K
_RPW = _ROWS // _NW
_CH = 64


def _sc_gather_body(table_ref, gidx_ref, out_ref, idx_v, rows_a, rows_b, sem):
    wid = lax.axis_index("s") * 2 + lax.axis_index("c")
    base = wid * _RPW
    bufs = (rows_a, rows_b)
    pltpu.sync_copy(gidx_ref.at[pl.ds(base, _CH)], idx_v)
    cp = pltpu.async_copy(table_ref.at[idx_v], bufs[0], sem)
    for c in range(_RPW // _CH):
        cp.wait()
        if c + 1 < _RPW // _CH:
            pltpu.sync_copy(gidx_ref.at[pl.ds(base + (c + 1) * _CH, _CH)],
                            idx_v)
            cp = pltpu.async_copy(table_ref.at[idx_v], bufs[(c + 1) % 2], sem)
        pltpu.sync_copy(bufs[c % 2], out_ref.at[pl.ds(base + c * _CH, _CH)])


_sc_gather = functools.partial(
    pl.kernel,
    mesh=plsc.VectorSubcoreMesh(core_axis_name="c", subcore_axis_name="s"),
    out_type=jax.ShapeDtypeStruct((_ROWS, _D), jnp.float32),
    scratch_types=[
        pltpu.VMEM((_CH,), jnp.int32),
        pltpu.VMEM((_CH, _D), jnp.float32),
        pltpu.VMEM((_CH, _D), jnp.float32),
        pltpu.SemaphoreType.DMA,
    ],
)(_sc_gather_body)


def _mlp_body(stats_ref, gath_ref, w_ref, W1_ref, b1_ref, W2_ref, b2_ref,
              g1_ref, be1_ref, lng_g_ref, lng_b_ref, Wg1_ref, Wg1s_ref,
              bg1_ref, wg2_ref, bg2_ref, l3g_ref, l3b_ref, W3_ref, b3_ref,
              ev_ref, raw_ref, usage_ref, semout_ref, ev_acc):
    i = pl.program_id(0)
    gath = gath_ref[...]
    gath = jnp.where(jnp.isfinite(gath), gath, 0.0)

    mu = jnp.mean(gath, axis=1, keepdims=True)
    var = jnp.mean((gath - mu) ** 2, axis=1, keepdims=True)
    xn = (gath - mu) / jnp.sqrt(var + 1e-5) * g1_ref[...] + be1_ref[...]
    h = _dot(xn.astype(jnp.bfloat16), W1_ref[...]) + b1_ref[...]
    h = _gelu(h)
    h = _dot(h.astype(jnp.bfloat16), W2_ref[...]) + b2_ref[...]

    wcol = w_ref[...]
    nrow = _GPB * _HZ * _K
    nseg = _GPB * _HZ
    r_io = lax.broadcasted_iota(jnp.int32, (nseg, nrow), 1)
    s_io = lax.broadcasted_iota(jnp.int32, (nseg, nrow), 0)
    seg = ((r_io // _K) == s_io).astype(jnp.float32)
    ev = _dot(seg, h * wcol)
    ev_ref[...] = ev
    ev_acc[pl.ds(i * nseg, nseg), :] = ev
    raw = _dot(seg, gath * wcol)
    n = jnp.sqrt(jnp.sum(raw * raw, axis=1, keepdims=True))
    raw_ref[...] = raw / jnp.clip(n, 1e-12)

    @pl.when(i == _STEPS - 1)
    def _gate():
        evf = ev_acc[...]
        st = stats_ref[...]
        width = float(_HID + 4)
        mu = (jnp.sum(evf, axis=1, keepdims=True) +
              jnp.sum(st, axis=1, keepdims=True)) / width
        var = (jnp.sum((evf - mu) ** 2, axis=1, keepdims=True) +
               jnp.sum((st - mu) ** 2, axis=1, keepdims=True)) / width
        inv = 1.0 / jnp.sqrt(var + 1e-5)
        evn = ((evf - mu) * inv * lng_g_ref[:, :_HID] + lng_b_ref[:, :_HID])
        stn = ((st - mu) * inv * lng_g_ref[:, _HID:] + lng_b_ref[:, _HID:])
        g = _dot(evn.astype(jnp.bfloat16), Wg1_ref[...]) + bg1_ref[...]
        for j in range(4):
            g = g + stn[:, j:j + 1] * Wg1s_ref[j:j + 1, :]
        g = _gelu(g)
        u = jnp.sum(g * wg2_ref[...], axis=1, keepdims=True) + bg2_ref[...]
        usage_ref[...] = jax.nn.sigmoid(u)

        mu3 = jnp.mean(evf, axis=1, keepdims=True)
        var3 = jnp.mean((evf - mu3) ** 2, axis=1, keepdims=True)
        evn3 = ((evf - mu3) / jnp.sqrt(var3 + 1e-5) * l3g_ref[...]
                + l3b_ref[...])
        s = _dot(evn3.astype(jnp.bfloat16), W3_ref[...]) + b3_ref[...]
        n3 = jnp.sqrt(jnp.sum(s * s, axis=1, keepdims=True))
        semout_ref[...] = s / jnp.clip(n3, 1e-12)


def _full(shape):
    return pl.BlockSpec(shape, lambda *_: tuple(0 for _ in shape))


@jax.jit
def _run(sem_flat, sel_flat, conf_g, agree_g,
         ln1_g, ln1_b, W1, b1, W2, b2,
         lng_g, lng_b, Wg1, bg1, Wg2, bg2,
         ln3_g, ln3_b, W3, b3):
    idx, gidx, w, stats = pl.pallas_call(
        _topk_body,
        out_shape=[
            jax.ShapeDtypeStruct((_R, _K), jnp.int32),
            jax.ShapeDtypeStruct((_R, _K), jnp.int32),
            jax.ShapeDtypeStruct((_R, _K), jnp.float32),
            jax.ShapeDtypeStruct((_R, 4), jnp.float32),
        ],
    )(sel_flat, conf_g, agree_g)

    gath = _sc_gather(sem_flat.reshape(_G * _T, _D), gidx.reshape(_ROWS))

    w_col = w.reshape(_R * _K, 1)
    ev, raw, usage, semantic = pl.pallas_call(
        _mlp_body,
        grid=(_STEPS,),
        in_specs=[
            _full((_R, 4)),
            pl.BlockSpec((_GPB * _HZ * _K, _D), lambda i: (i, 0)),
            pl.BlockSpec((_GPB * _HZ * _K, 1), lambda i: (i, 0)),
            _full((_D, _HID)),
            _full((1, _HID)),
            _full((_HID, _HID)),
            _full((1, _HID)),
            _full((1, _D)),
            _full((1, _D)),
            _full((1, _HID + 4)),
            _full((1, _HID + 4)),
            _full((_HID, _HID)),
            _full((4, _HID)),
            _full((1, _HID)),
            _full((1, _HID)),
            _full((1, 1)),
            _full((1, _HID)),
            _full((1, _HID)),
            _full((_HID, _D)),
            _full((1, _D)),
        ],
        out_specs=[
            pl.BlockSpec((_GPB * _HZ, _HID), lambda i: (i, 0)),
            pl.BlockSpec((_GPB * _HZ, _D), lambda i: (i, 0)),
            _full((_R, 1)),
            _full((_R, _D)),
        ],
        out_shape=[
            jax.ShapeDtypeStruct((_R, _HID), jnp.float32),
            jax.ShapeDtypeStruct((_R, _D), jnp.float32),
            jax.ShapeDtypeStruct((_R, 1), jnp.float32),
            jax.ShapeDtypeStruct((_R, _D), jnp.float32),
        ],
        scratch_shapes=[pltpu.VMEM((_R, _HID), jnp.float32)],
        compiler_params=pltpu.CompilerParams(
            dimension_semantics=("arbitrary",)),
    )(stats, gath, w_col,
      W1.astype(jnp.bfloat16), b1.reshape(1, _HID),
      W2.astype(jnp.bfloat16), b2.reshape(1, _HID),
      ln1_g.reshape(1, _D), ln1_b.reshape(1, _D),
      lng_g.reshape(1, _HID + 4), lng_b.reshape(1, _HID + 4),
      Wg1[:_HID].astype(jnp.bfloat16), Wg1[_HID:], bg1.reshape(1, _HID),
      Wg2.reshape(1, _HID), bg2.reshape(1, 1),
      ln3_g.reshape(1, _HID), ln3_b.reshape(1, _HID),
      W3.astype(jnp.bfloat16), b3.reshape(1, _D))

    return idx, w, stats, gath, ev, raw, usage, semantic


def kernel(obs_semantic_measurements, obs_measurement_confidence,
           teacher_agreement_score, selector_weight,
           ln1_g, ln1_b, W1, b1, W2, b2,
           lng_g, lng_b, Wg1, bg1, Wg2, bg2,
           ln3_g, ln3_b, W3, b3):
    sem = obs_semantic_measurements.astype(jnp.float32)
    conf = obs_measurement_confidence.astype(jnp.float32)
    agree = teacher_agreement_score.astype(jnp.float32)
    sel = selector_weight.astype(jnp.float32)

    sem_flat = sem.reshape(_G, _T, _D)
    sel_flat = sel.reshape(_R, _T)
    conf_g = jnp.broadcast_to(conf.reshape(_G, 1, _T),
                              (_G, _HZ, _T)).reshape(_R, _T)
    agree_g = jnp.broadcast_to(agree.reshape(_G, 1, _T),
                               (_G, _HZ, _T)).reshape(_R, _T)

    idx, w, stats, gath, ev, raw, usage, semantic = _run(
        sem_flat, sel_flat, conf_g, agree_g,
        ln1_g, ln1_b, W1, b1, W2, b2,
        lng_g, lng_b, Wg1, bg1, Wg2, bg2,
        ln3_g, ln3_b, W3, b3)

    idx_o = idx.reshape(_B, _M, _HZ, _K)
    w_o = w.reshape(_B, _M, _HZ, _K)
    gath_o = gath.reshape(_B, _M, _HZ, _K, _D)
    ev_o = ev.reshape(_B, _M, _HZ, _HID)
    sem_o = semantic.reshape(_B, _M, _HZ, _D)
    raw_o = raw.reshape(_B, _M, _HZ, _D)
    usage_o = usage.reshape(_B, _M, _HZ)
    ent_o = stats[:, 1].reshape(_B, _M, _HZ)
    maxw_o = stats[:, 0].reshape(_B, _M, _HZ)
    return (idx_o, w_o, gath_o, ev_o, sem_o, raw_o, usage_o, ent_o, maxw_o)


# speedup vs baseline: 1.1022x; 1.1022x over previous
import functools
import math

import jax
import jax.numpy as jnp
from jax import lax
from jax.experimental import pallas as pl
from jax.experimental.pallas import tpu as pltpu
from jax.experimental.pallas import tpu_sc as plsc

_B, _M, _T, _HZ = 32, 8, 128, 4
_D, _HID, _K = 512, 1024, 8
_G = _B * _M
_R = _G * _HZ
_GPB = 32
_STEPS = _G // _GPB
_RC = 256


def _dot(a, b):
    return jax.lax.dot_general(a, b, (((1,), (0,)), ((), ())),
                               preferred_element_type=jnp.float32)


def _gelu(x):
    return 0.5 * x * (1.0 + lax.erf(x * (1.0 / math.sqrt(2.0))))


def _topk_body(sel_ref, conf_ref, agree_ref, idx_ref, gidx_ref, w_ref,
               stats_ref):
    sel = sel_ref[...]
    conf = jnp.clip(conf_ref[...], 0.0, 1.0)
    agree = jnp.clip(agree_ref[...], 0.0, 1.0)
    iota = lax.broadcasted_iota(jnp.int32, (_R, _T), 1)
    row = sel
    idxs, vals, confs, agrees = [], [], [], []
    for _ in range(_K):
        m = jnp.max(row, axis=1, keepdims=True)
        is_max = row == m
        idx_j = jnp.min(jnp.where(is_max, iota, _T), axis=1, keepdims=True)
        pick = iota == idx_j
        confs.append(jnp.sum(jnp.where(pick, conf, 0.0), axis=1,
                             keepdims=True))
        agrees.append(jnp.sum(jnp.where(pick, agree, 0.0), axis=1,
                              keepdims=True))
        row = jnp.where(pick, -jnp.inf, row)
        idxs.append(idx_j)
        vals.append(m)
    idx = jnp.concatenate(idxs, axis=1)
    val = jnp.concatenate(vals, axis=1)
    tconf = jnp.concatenate(confs, axis=1)
    tagree = jnp.concatenate(agrees, axis=1)
    w = val * jnp.clip(tconf, 0.05, 1.0) * jnp.clip(tagree, 0.05, 1.0)
    w = w / jnp.clip(jnp.sum(w, axis=1, keepdims=True), 1e-6)
    maxw = jnp.max(w, axis=1, keepdims=True)
    wc = jnp.clip(w, 1e-8)
    ent = (-jnp.sum(wc * jnp.log(wc), axis=1, keepdims=True)
           / math.log(float(max(_K, 2))))
    cm = jnp.sum(w * tconf, axis=1, keepdims=True)
    am = jnp.sum(w * tagree, axis=1, keepdims=True)
    idx_ref[...] = idx
    r_io = lax.broadcasted_iota(jnp.int32, (_R, _K), 0)
    gidx_ref[...] = idx + (r_io // _HZ) * _T
    w_ref[...] = w
    stats_ref[...] = jnp.concatenate([maxw, ent, cm, am], axis=1)


_NW = 32
_ROWS = _R * _K
_RPW = _ROWS // _NW
_CH = 64


def _sc_gather_body(table_ref, gidx_ref, out_ref, idx_v, rows_a, rows_b, sem):
    wid = lax.axis_index("s") * 2 + lax.axis_index("c")
    base = wid * _RPW
    bufs = (rows_a, rows_b)
    pltpu.sync_copy(gidx_ref.at[pl.ds(base, _CH)], idx_v)
    cp = pltpu.async_copy(table_ref.at[idx_v], bufs[0], sem)
    for c in range(_RPW // _CH):
        cp.wait()
        if c + 1 < _RPW // _CH:
            pltpu.sync_copy(gidx_ref.at[pl.ds(base + (c + 1) * _CH, _CH)],
                            idx_v)
            cp = pltpu.async_copy(table_ref.at[idx_v], bufs[(c + 1) % 2], sem)
        pltpu.sync_copy(bufs[c % 2], out_ref.at[pl.ds(base + c * _CH, _CH)])


_sc_gather = functools.partial(
    pl.kernel,
    mesh=plsc.VectorSubcoreMesh(core_axis_name="c", subcore_axis_name="s"),
    out_type=jax.ShapeDtypeStruct((_ROWS, _D), jnp.float32),
    scratch_types=[
        pltpu.VMEM((_CH,), jnp.int32),
        pltpu.VMEM((_CH, _D), jnp.float32),
        pltpu.VMEM((_CH, _D), jnp.float32),
        pltpu.SemaphoreType.DMA,
    ],
)(_sc_gather_body)


def _mlp_body(stats_ref, gath_ref, w_ref, W1_ref, b1_ref, W2_ref, b2_ref,
              g1_ref, be1_ref, lng_g_ref, lng_b_ref, Wg1_ref, Wg1s_ref,
              bg1_ref, wg2_ref, bg2_ref, l3g_ref, l3b_ref, W3_ref, b3_ref,
              ev_ref, raw_ref, usage_ref, semout_ref, ev_acc):
    i = pl.program_id(0)
    gath = gath_ref[...]
    gath = jnp.where(jnp.isfinite(gath), gath, 0.0)

    mu = jnp.mean(gath, axis=1, keepdims=True)
    var = jnp.mean((gath - mu) ** 2, axis=1, keepdims=True)
    xn = (gath - mu) / jnp.sqrt(var + 1e-5) * g1_ref[...] + be1_ref[...]
    h = _dot(xn, W1_ref[...]) + b1_ref[...]
    h = _gelu(h)
    h = _dot(h, W2_ref[...]) + b2_ref[...]

    wcol = w_ref[...]
    nrow = _GPB * _HZ * _K
    nseg = _GPB * _HZ
    r_io = lax.broadcasted_iota(jnp.int32, (nseg, nrow), 1)
    s_io = lax.broadcasted_iota(jnp.int32, (nseg, nrow), 0)
    seg = ((r_io // _K) == s_io).astype(jnp.float32)
    ev = _dot(seg, h * wcol)
    ev_ref[...] = ev
    ev_acc[pl.ds(i * nseg, nseg), :] = ev
    raw = _dot(seg, gath * wcol)
    n = jnp.sqrt(jnp.sum(raw * raw, axis=1, keepdims=True))
    raw_ref[...] = raw / jnp.clip(n, 1e-12)

    @pl.when(i == _STEPS - 1)
    def _gate():
        evf = ev_acc[...]
        st = stats_ref[...]
        width = float(_HID + 4)
        mu = (jnp.sum(evf, axis=1, keepdims=True) +
              jnp.sum(st, axis=1, keepdims=True)) / width
        var = (jnp.sum((evf - mu) ** 2, axis=1, keepdims=True) +
               jnp.sum((st - mu) ** 2, axis=1, keepdims=True)) / width
        inv = 1.0 / jnp.sqrt(var + 1e-5)
        evn = ((evf - mu) * inv * lng_g_ref[:, :_HID] + lng_b_ref[:, :_HID])
        stn = ((st - mu) * inv * lng_g_ref[:, _HID:] + lng_b_ref[:, _HID:])
        g = _dot(evn, Wg1_ref[...]) + bg1_ref[...]
        for j in range(4):
            g = g + stn[:, j:j + 1] * Wg1s_ref[j:j + 1, :]
        g = _gelu(g)
        u = jnp.sum(g * wg2_ref[...], axis=1, keepdims=True) + bg2_ref[...]
        usage_ref[...] = jax.nn.sigmoid(u)

        mu3 = jnp.mean(evf, axis=1, keepdims=True)
        var3 = jnp.mean((evf - mu3) ** 2, axis=1, keepdims=True)
        evn3 = ((evf - mu3) / jnp.sqrt(var3 + 1e-5) * l3g_ref[...]
                + l3b_ref[...])
        s = _dot(evn3, W3_ref[...]) + b3_ref[...]
        n3 = jnp.sqrt(jnp.sum(s * s, axis=1, keepdims=True))
        semout_ref[...] = s / jnp.clip(n3, 1e-12)


def _full(shape):
    return pl.BlockSpec(shape, lambda *_: tuple(0 for _ in shape))


@jax.jit
def _run(sem_flat, sel_flat, conf_g, agree_g,
         ln1_g, ln1_b, W1, b1, W2, b2,
         lng_g, lng_b, Wg1, bg1, Wg2, bg2,
         ln3_g, ln3_b, W3, b3):
    idx, gidx, w, stats = pl.pallas_call(
        _topk_body,
        out_shape=[
            jax.ShapeDtypeStruct((_R, _K), jnp.int32),
            jax.ShapeDtypeStruct((_R, _K), jnp.int32),
            jax.ShapeDtypeStruct((_R, _K), jnp.float32),
            jax.ShapeDtypeStruct((_R, 4), jnp.float32),
        ],
    )(sel_flat, conf_g, agree_g)

    gath = _sc_gather(sem_flat.reshape(_G * _T, _D), gidx.reshape(_ROWS))

    w_col = w.reshape(_R * _K, 1)
    ev, raw, usage, semantic = pl.pallas_call(
        _mlp_body,
        grid=(_STEPS,),
        in_specs=[
            _full((_R, 4)),
            pl.BlockSpec((_GPB * _HZ * _K, _D), lambda i: (i, 0)),
            pl.BlockSpec((_GPB * _HZ * _K, 1), lambda i: (i, 0)),
            _full((_D, _HID)),
            _full((1, _HID)),
            _full((_HID, _HID)),
            _full((1, _HID)),
            _full((1, _D)),
            _full((1, _D)),
            _full((1, _HID + 4)),
            _full((1, _HID + 4)),
            _full((_HID, _HID)),
            _full((4, _HID)),
            _full((1, _HID)),
            _full((1, _HID)),
            _full((1, 1)),
            _full((1, _HID)),
            _full((1, _HID)),
            _full((_HID, _D)),
            _full((1, _D)),
        ],
        out_specs=[
            pl.BlockSpec((_GPB * _HZ, _HID), lambda i: (i, 0)),
            pl.BlockSpec((_GPB * _HZ, _D), lambda i: (i, 0)),
            _full((_R, 1)),
            _full((_R, _D)),
        ],
        out_shape=[
            jax.ShapeDtypeStruct((_R, _HID), jnp.float32),
            jax.ShapeDtypeStruct((_R, _D), jnp.float32),
            jax.ShapeDtypeStruct((_R, 1), jnp.float32),
            jax.ShapeDtypeStruct((_R, _D), jnp.float32),
        ],
        scratch_shapes=[pltpu.VMEM((_R, _HID), jnp.float32)],
        compiler_params=pltpu.CompilerParams(
            dimension_semantics=("arbitrary",)),
    )(stats, gath, w_col,
      W1, b1.reshape(1, _HID),
      W2, b2.reshape(1, _HID),
      ln1_g.reshape(1, _D), ln1_b.reshape(1, _D),
      lng_g.reshape(1, _HID + 4), lng_b.reshape(1, _HID + 4),
      Wg1[:_HID], Wg1[_HID:], bg1.reshape(1, _HID),
      Wg2.reshape(1, _HID), bg2.reshape(1, 1),
      ln3_g.reshape(1, _HID), ln3_b.reshape(1, _HID),
      W3, b3.reshape(1, _D))

    return idx, w, stats, gath, ev, raw, usage, semantic


def kernel(obs_semantic_measurements, obs_measurement_confidence,
           teacher_agreement_score, selector_weight,
           ln1_g, ln1_b, W1, b1, W2, b2,
           lng_g, lng_b, Wg1, bg1, Wg2, bg2,
           ln3_g, ln3_b, W3, b3):
    sem = obs_semantic_measurements.astype(jnp.float32)
    conf = obs_measurement_confidence.astype(jnp.float32)
    agree = teacher_agreement_score.astype(jnp.float32)
    sel = selector_weight.astype(jnp.float32)

    sem_flat = sem.reshape(_G, _T, _D)
    sel_flat = sel.reshape(_R, _T)
    conf_g = jnp.broadcast_to(conf.reshape(_G, 1, _T),
                              (_G, _HZ, _T)).reshape(_R, _T)
    agree_g = jnp.broadcast_to(agree.reshape(_G, 1, _T),
                               (_G, _HZ, _T)).reshape(_R, _T)

    idx, w, stats, gath, ev, raw, usage, semantic = _run(
        sem_flat, sel_flat, conf_g, agree_g,
        ln1_g, ln1_b, W1, b1, W2, b2,
        lng_g, lng_b, Wg1, bg1, Wg2, bg2,
        ln3_g, ln3_b, W3, b3)

    idx_o = idx.reshape(_B, _M, _HZ, _K)
    w_o = w.reshape(_B, _M, _HZ, _K)
    gath_o = gath.reshape(_B, _M, _HZ, _K, _D)
    ev_o = ev.reshape(_B, _M, _HZ, _HID)
    sem_o = semantic.reshape(_B, _M, _HZ, _D)
    raw_o = raw.reshape(_B, _M, _HZ, _D)
    usage_o = usage.reshape(_B, _M, _HZ)
    ent_o = stats[:, 1].reshape(_B, _M, _HZ)
    maxw_o = stats[:, 0].reshape(_B, _M, _HZ)
    return (idx_o, w_o, gath_o, ev_o, sem_o, raw_o, usage_o, ent_o, maxw_o)
